# baseline (speedup 1.0000x reference)
import functools
import math

import jax
import jax.numpy as jnp
from jax import lax
from jax.experimental import pallas as pl
from jax.experimental.pallas import tpu as pltpu
from jax.experimental.pallas import tpu_sc as plsc

_B = 2
_N = 2048
_FEAT = 15
_AA = 24
_KS = (2, 3, 4, 5, 6, 7)
_FN = 32
_AFN = _FN * len(_KS)
_H = 4
_HID = 128
_DH = 32
_TOPK = 102
_PMAX = _AA
_CCOLS = len(_KS) * _PMAX * _FN
_XCOLS = _FEAT * _AA + 1
_XPAD = 384


def _build_conv_matrix(cws, cbs):
    blocks = []
    bias_cols = []
    for g, h in enumerate(_KS):
        w = cws[g].transpose(1, 2, 0)
        cols = []
        np_valid = _AA - h + 1
        for p in range(_PMAX):
            if p < np_valid:
                m = jnp.zeros((_FEAT, _AA, _FN), jnp.float32)
                m = lax.dynamic_update_slice(m, w, (0, p, 0))
                m = m.reshape(_FEAT * _AA, _FN)
                bias_cols.append(cbs[g])
            else:
                m = jnp.zeros((_FEAT * _AA, _FN), jnp.float32)
                bias_cols.append(jnp.full((_FN,), -1e30))
            cols.append(m)
        blocks.append(jnp.concatenate(cols, axis=1))
    W = jnp.concatenate(blocks, axis=1)
    biasmask = jnp.concatenate(bias_cols)[None, :]
    return jnp.pad(W, ((0, _XPAD - (_XCOLS - 1)), (0, 0))), biasmask


def _featqkv_body(x_ref, wc_ref, bmask_ref, wqkv_ref, bqkv_ref, out_ref):
    acts = jnp.maximum(
        jnp.dot(x_ref[...], wc_ref[...], preferred_element_type=jnp.float32)
        + bmask_ref[...], 0.0)
    bm = acts.shape[0]
    feats = jnp.max(acts.reshape(bm, len(_KS), _PMAX, _FN), axis=2)
    feats = feats.reshape(bm, _AFN)
    out_ref[...] = (
        jnp.dot(feats, wqkv_ref[...], preferred_element_type=jnp.float32)
        + bqkv_ref[...])


def _scores_body(q_ref, k_ref, out_ref):
    q = q_ref[0]
    k = k_ref[0]
    outs = []
    for h in range(_H):
        qh = q[:, h * _DH:(h + 1) * _DH]
        kh = k[:, h * _DH:(h + 1) * _DH]
        outs.append(
            jax.lax.dot_general(qh, kh, (((1,), (1,)), ((), ())),
                                preferred_element_type=jnp.float32)
            / math.sqrt(_DH))
    out_ref[0] = jnp.concatenate(outs, axis=1)


def _pooled_body(v_ref, wa_ref, ba_ref, wm_ref, bm_ref, out_ref):
    vmean = jnp.sum(v_ref[...], axis=1) / _N
    ctx = jnp.dot(vmean, wa_ref[...], preferred_element_type=jnp.float32) + ba_ref[...]
    logits = jnp.dot(ctx, wm_ref[...], preferred_element_type=jnp.float32) + bm_ref[...]
    padded = jnp.concatenate([logits, jnp.zeros((_B, 126), jnp.float32)], axis=1)
    out_ref[...] = jnp.concatenate(
        [padded, jnp.zeros((8 - _B, 128), jnp.float32)], axis=0)


_NROWS = _B * _N
_NW = 32
_RPW = _NROWS // _NW
_OPAD = 128


def _sorted_topk_vregs(ks, vs, desc):
    n = len(ks)
    if n == 1:
        k, v = plsc.sort_key_val(ks[0], vs[0], descending=desc)
        return [k], [v]
    ak, av = _sorted_topk_vregs(ks[:n // 2], vs[:n // 2], desc)
    bk, bv = _sorted_topk_vregs(ks[n // 2:], vs[n // 2:], not desc)
    return _bitonic_merge(ak + bk, av + bv, desc)


def _bitonic_merge(ks, vs, desc):
    n = len(ks)
    if n == 1:
        k, v = plsc.sort_key_val(ks[0], vs[0], descending=desc)
        return [k], [v]
    half = n // 2
    ks = list(ks)
    vs = list(vs)
    for i in range(half):
        a_k, b_k = ks[i], ks[i + half]
        a_v, b_v = vs[i], vs[i + half]
        m = (a_k >= b_k) if desc else (a_k <= b_k)
        ks[i] = jnp.where(m, a_k, b_k)
        ks[i + half] = jnp.where(m, b_k, a_k)
        vs[i] = jnp.where(m, a_v, b_v)
        vs[i + half] = jnp.where(m, b_v, a_v)
    k1, v1 = _bitonic_merge(ks[:half], vs[:half], desc)
    k2, v2 = _bitonic_merge(ks[half:], vs[half:], desc)
    return k1 + k2, v1 + v2


def _route_body(s4_hbm, out_hbm, s4row, keys, hist1k, cge1k, ckey, cidx,
                hist32, tkk, tki, eqb, outrow, sgbuf, dmasem):
    cid = lax.axis_index("c")
    sid = lax.axis_index("s")
    wid = sid * 2 + cid
    row0 = wid * _RPW
    iota = lax.broadcasted_iota(jnp.int32, (16,), 0)
    ones_i = jnp.ones((16,), jnp.int32)
    zeros_i = jnp.zeros((16,), jnp.int32)
    NV = _N // 16
    NB = 256

    pltpu.async_copy(s4_hbm.at[row0], s4row, dmasem)

    def row_body(r, _c):
        row = row0 + r
        pltpu.make_async_copy(s4_hbm.at[row], s4row, dmasem).wait()

        for v in range(NB // 16):
            hist1k[pl.ds(v * 16, 16)] = zeros_i

        def pass_a(i, c):
            for j in range(4):
                o = i * 64 + j * 16
                s = (s4row[pl.ds(o, 16)] + s4row[pl.ds(_N + o, 16)]
                     + s4row[pl.ds(2 * _N + o, 16)]
                     + s4row[pl.ds(3 * _N + o, 16)])
                b = plsc.bitcast(s, jnp.uint32)
                neg = b >= jnp.uint32(0x80000000)
                key = jnp.where(neg, ~b, b | jnp.uint32(0x80000000))
                keys[pl.ds(o, 16)] = plsc.bitcast(key, jnp.int32)
                dig = (key >> 24).astype(jnp.int32)
                plsc.addupdate_scatter(hist1k, [dig], ones_i)
            return c
        lax.fori_loop(0, NV // 4, pass_a, 0)

        def suf(vi, carry):
            v = NB // 16 - 1 - vi
            h = hist1k[pl.ds(v * 16, 16)]
            s = lax.rev(plsc.cumsum(lax.rev(h, (0,))), (0,))
            cge1k[pl.ds(v * 16, 16)] = s + carry
            return carry + jnp.sum(h)
        lax.fori_loop(0, NB // 16, suf, 0)

        def cnt_ge(vi, acc):
            c = cge1k[pl.ds(vi * 16, 16)]
            return acc + (c >= _TOPK).astype(jnp.int32)
        g = jnp.sum(lax.fori_loop(0, NB // 16, cnt_ge, zeros_i)) - 1
        gp = jnp.minimum(g + 1, NB - 1)
        m_gt = jnp.where(
            g >= NB - 1, 0,
            plsc.load_gather(cge1k, [jnp.broadcast_to(gp, (16,))]))

        def pass_b(i, carry):
            off_hi, off_eq = carry
            for j in range(2):
                o = i * 32 + j * 16
                key = plsc.bitcast(keys[pl.ds(o, 16)], jnp.uint32)
                top = (key >> 24).astype(jnp.int32)
                idxv = o + iota
                m_hi = top > g
                m_eq = top == g
                pos_hi = plsc.cumsum(ones_i, mask=m_hi) - 1 + off_hi
                pos_hi = jnp.where(m_hi, pos_hi, 0)
                plsc.store_scatter(tkk, [pos_hi],
                                   plsc.bitcast(key, jnp.int32), mask=m_hi)
                plsc.store_scatter(tki, [pos_hi], idxv, mask=m_hi)
                pos_eq = plsc.cumsum(ones_i, mask=m_eq) - 1 + off_eq
                pos_eq = jnp.where(m_eq, pos_eq, 0)
                plsc.store_scatter(ckey, [pos_eq],
                                   plsc.bitcast(key, jnp.int32), mask=m_eq)
                plsc.store_scatter(cidx, [pos_eq], idxv, mask=m_eq)
                off_hi = off_hi + plsc.all_reduce_population_count(m_hi)
                off_eq = off_eq + plsc.all_reduce_population_count(m_eq)
            return (off_hi, off_eq)
        _, ccount_v = lax.fori_loop(0, NV // 2, pass_b, (zeros_i, zeros_i))
        ccount = jnp.max(ccount_v)
        ctiles = (ccount + 15) // 16
        need = _TOPK - m_gt
        prefix = jnp.broadcast_to(g.astype(jnp.uint32), (16,))

        for shift, width in ((19, 5), (14, 5), (9, 5), (4, 5), (0, 4)):
            nb = 1 << width
            topshift = shift + width
            hist32[pl.ds(0, 16)] = zeros_i
            hist32[pl.ds(16, 16)] = zeros_i

            def ph(t, c, shift=shift, topshift=topshift, nb=nb, prefix=prefix):
                key = plsc.bitcast(ckey[pl.ds(t * 16, 16)], jnp.uint32)
                valid = (t * 16 + iota) < ccount
                alive = jnp.logical_and((key >> topshift) == prefix, valid)
                dig = ((key >> shift).astype(jnp.int32)) & (nb - 1)
                plsc.addupdate_scatter(hist32, [dig], ones_i, mask=alive)
                return c
            lax.fori_loop(0, ctiles, ph, 0)
            h_lo = hist32[pl.ds(0, 16)]
            h_hi = hist32[pl.ds(16, 16)]
            s_hi = lax.rev(plsc.cumsum(lax.rev(h_hi, (0,))), (0,))
            s_lo = (lax.rev(plsc.cumsum(lax.rev(h_lo, (0,))), (0,))
                    + jnp.broadcast_to(jnp.sum(h_hi), (16,)))
            d = jnp.sum((s_lo >= need).astype(jnp.int32)
                        + (s_hi >= need).astype(jnp.int32)) - 1
            m_gt2 = jnp.sum(jnp.where(iota > d, h_lo, 0)
                            + jnp.where(iota + 16 > d, h_hi, 0))
            need = need - m_gt2
            prefix = (prefix << width) | jnp.broadcast_to(
                d.astype(jnp.uint32), (16,))
        t_key = prefix

        def sel(t, off):
            key = plsc.bitcast(ckey[pl.ds(t * 16, 16)], jnp.uint32)
            idxv = cidx[pl.ds(t * 16, 16)]
            valid = (t * 16 + iota) < ccount
            m3 = jnp.logical_and(key > t_key, valid)
            pos = plsc.cumsum(ones_i, mask=m3) - 1 + off
            pos = jnp.where(m3, pos, 0)
            plsc.store_scatter(tkk, [pos], plsc.bitcast(key, jnp.int32), mask=m3)
            plsc.store_scatter(tki, [pos], idxv, mask=m3)
            return off + plsc.all_reduce_population_count(m3)
        off2 = lax.fori_loop(0, ctiles, sel, m_gt)

        def eqf(t, off):
            key = plsc.bitcast(ckey[pl.ds(t * 16, 16)], jnp.uint32)
            idxv = cidx[pl.ds(t * 16, 16)]
            valid = (t * 16 + iota) < ccount
            m4 = jnp.logical_and(key == t_key, valid)
            pos = plsc.cumsum(ones_i, mask=m4) - 1 + off
            m4c = jnp.logical_and(m4, pos < 128)
            pos = jnp.where(m4c, pos, 0)
            plsc.store_scatter(eqb, [pos], idxv, mask=m4c)
            return off + plsc.all_reduce_population_count(m4)
        lax.fori_loop(0, ctiles, eqf, zeros_i)
        for j in range(7):
            e = eqb[pl.ds(j * 16, 16)]
            pos = off2 + j * 16 + iota
            m5 = pos < _TOPK
            posc = jnp.where(m5, pos, 0)
            plsc.store_scatter(tki, [posc], e, mask=m5)
            plsc.store_scatter(tkk, [posc], plsc.bitcast(t_key, jnp.int32), mask=m5)

        v6k = tkk[pl.ds(96, 16)]
        v6i = tki[pl.ds(96, 16)]
        mpad = (96 + iota) >= _TOPK
        tkk[pl.ds(96, 16)] = jnp.where(mpad, 0, v6k)
        tki[pl.ds(96, 16)] = jnp.where(mpad, 0, v6i)
        tkk[pl.ds(112, 16)] = zeros_i
        tki[pl.ds(112, 16)] = zeros_i

        kregs = [plsc.bitcast(tkk[pl.ds(j * 16, 16)], jnp.uint32)
                 for j in range(8)]
        iregs = [tki[pl.ds(j * 16, 16)] for j in range(8)]
        sk, si = _sorted_topk_vregs(kregs, iregs, True)

        for h in range(_H):
            for j in range(7):
                sgbuf[pl.ds(h * 112 + j * 16, 16)] = plsc.load_gather(
                    s4row, [si[j] + h * _N])
        nxt = jnp.minimum(row + 1, row0 + _RPW - 1)
        pltpu.async_copy(s4_hbm.at[nxt], s4row, dmasem)

        for h in range(_H):
            gs = [sgbuf[pl.ds(h * 112 + j * 16, 16)] for j in range(7)]
            lanes = [j * 16 + iota for j in range(7)]
            gm = [jnp.where(lanes[j] < _TOPK, gs[j], -1e30) for j in range(7)]
            mx = gm[0]
            for j in range(1, 7):
                mx = jnp.maximum(mx, gm[j])
            mxs = jnp.max(mx)
            es = [jnp.where(lanes[j] < _TOPK,
                            jnp.exp(gs[j] - mxs), 0.0) for j in range(7)]
            tot = es[0]
            for j in range(1, 7):
                tot = tot + es[j]
            ssum = jnp.sum(tot)
            for j in range(7):
                outrow[pl.ds(h * _OPAD + j * 16, 16)] = es[j] / ssum
            outrow[pl.ds(h * _OPAD + 112, 16)] = jnp.zeros((16,), jnp.float32)

        pltpu.sync_copy(outrow, out_hbm.at[row])
        return _c

    lax.fori_loop(0, _RPW, row_body, 0)
    pltpu.make_async_copy(
        s4_hbm.at[row0 + _RPW - 1], s4row, dmasem).wait()


def _make_route_kernel():
    mesh = plsc.VectorSubcoreMesh(core_axis_name="c", subcore_axis_name="s")
    return pl.kernel(
        _route_body,
        out_type=jax.ShapeDtypeStruct((_NROWS, _H * _OPAD), jnp.float32),
        mesh=mesh,
        compiler_params=pltpu.CompilerParams(needs_layout_passes=False),
        scratch_types=[
            pltpu.VMEM((_H * _N,), jnp.float32),
            pltpu.VMEM((_N,), jnp.int32),
            pltpu.VMEM((1024,), jnp.int32),
            pltpu.VMEM((1024,), jnp.int32),
            pltpu.VMEM((_N,), jnp.int32),
            pltpu.VMEM((_N,), jnp.int32),
            pltpu.VMEM((32,), jnp.int32),
            pltpu.VMEM((128,), jnp.int32),
            pltpu.VMEM((128,), jnp.int32),
            pltpu.VMEM((128,), jnp.int32),
            pltpu.VMEM((_H * _OPAD,), jnp.float32),
            pltpu.VMEM((_H * 112,), jnp.float32),
            pltpu.SemaphoreType.DMA,
        ],
    )


def kernel(x, cw2, cb2, cw3, cb3, cw4, cb4, cw5, cb5, cw6, cb6, cw7, cb7,
           Wq, bq, Wk, bk, Wv, bv, Wa, ba, Wm, bm):
    NR = _B * _N
    xf = x.reshape(NR, _FEAT * _AA)
    xa = jnp.concatenate(
        [xf, jnp.zeros((NR, _XPAD - (_XCOLS - 1)), jnp.float32)], axis=1)
    Wc, biasmask = _build_conv_matrix((cw2, cw3, cw4, cw5, cw6, cw7),
                                      (cb2, cb3, cb4, cb5, cb6, cb7))
    Wqkv = jnp.concatenate([Wq.T, Wk.T, Wv.T], axis=1)
    bqkv = jnp.concatenate([bq, bk, bv])[None, :]

    BM = 512
    qkv = pl.pallas_call(
        _featqkv_body,
        grid=(NR // BM,),
        in_specs=[
            pl.BlockSpec((BM, _XPAD), lambda i: (i, 0)),
            pl.BlockSpec((_XPAD, _CCOLS), lambda i: (0, 0)),
            pl.BlockSpec((1, _CCOLS), lambda i: (0, 0)),
            pl.BlockSpec((_AFN, 3 * _HID), lambda i: (0, 0)),
            pl.BlockSpec((1, 3 * _HID), lambda i: (0, 0)),
        ],
        out_specs=pl.BlockSpec((BM, 3 * _HID), lambda i: (i, 0)),
        out_shape=jax.ShapeDtypeStruct((NR, 3 * _HID), jnp.float32),
    )(xa, Wc, biasmask, Wqkv, bqkv)

    q = qkv[:, :_HID].reshape(_B, _N, _HID)
    k = qkv[:, _HID:2 * _HID].reshape(_B, _N, _HID)
    v = qkv[:, 2 * _HID:].reshape(_B, _N, _HID)

    BS = 256
    s4 = pl.pallas_call(
        _scores_body,
        grid=(_B, _N // BS),
        in_specs=[
            pl.BlockSpec((1, BS, _HID), lambda b, i: (b, i, 0)),
            pl.BlockSpec((1, _N, _HID), lambda b, i: (b, 0, 0)),
        ],
        out_specs=pl.BlockSpec((1, BS, _H * _N), lambda b, i: (b, i, 0)),
        out_shape=jax.ShapeDtypeStruct((_B, _N, _H * _N), jnp.float32),
    )(q, k)

    probs_pad = _make_route_kernel()(s4.reshape(_NROWS, _H * _N))
    probs = probs_pad.reshape(_B, _N, _H, _OPAD)[:, :, :, :_TOPK]
    probs_e = probs.transpose(0, 2, 1, 3)[:, :, :, None, :]

    pooled_pad = pl.pallas_call(
        _pooled_body,
        in_specs=[
            pl.BlockSpec((_B, _N, _HID), lambda: (0, 0, 0)),
            pl.BlockSpec((_HID, _AFN), lambda: (0, 0)),
            pl.BlockSpec((1, _AFN), lambda: (0, 0)),
            pl.BlockSpec((_AFN, 2), lambda: (0, 0)),
            pl.BlockSpec((1, 2), lambda: (0, 0)),
        ],
        out_specs=pl.BlockSpec((8, 128), lambda: (0, 0)),
        out_shape=jax.ShapeDtypeStruct((8, 128), jnp.float32),
    )(v, Wa.T, ba[None, :], Wm.T, bm[None, :])
    pooled = pooled_pad[:_B, :2]

    return pooled, probs_e


# speedup vs baseline: 1.3398x; 1.3398x over previous
import functools
import math

import jax
import jax.numpy as jnp
from jax import lax
from jax.experimental import pallas as pl
from jax.experimental.pallas import tpu as pltpu
from jax.experimental.pallas import tpu_sc as plsc

_B = 2
_N = 2048
_FEAT = 15
_AA = 24
_KS = (2, 3, 4, 5, 6, 7)
_FN = 32
_AFN = _FN * len(_KS)
_H = 4
_HID = 128
_DH = 32
_TOPK = 102
_PMAX = _AA
_CCOLS = len(_KS) * _PMAX * _FN
_XCOLS = _FEAT * _AA + 1
_XPAD = 384


def _build_conv_matrix(cws, cbs):
    blocks = []
    bias_cols = []
    for g, h in enumerate(_KS):
        w = cws[g].transpose(1, 2, 0)
        cols = []
        np_valid = _AA - h + 1
        for p in range(_PMAX):
            if p < np_valid:
                m = jnp.zeros((_FEAT, _AA, _FN), jnp.float32)
                m = lax.dynamic_update_slice(m, w, (0, p, 0))
                m = m.reshape(_FEAT * _AA, _FN)
                bias_cols.append(cbs[g])
            else:
                m = jnp.zeros((_FEAT * _AA, _FN), jnp.float32)
                bias_cols.append(jnp.full((_FN,), -1e30))
            cols.append(m)
        blocks.append(jnp.concatenate(cols, axis=1))
    W = jnp.concatenate(blocks, axis=1)
    biasmask = jnp.concatenate(bias_cols)[None, :]
    return jnp.pad(W, ((0, _XPAD - (_XCOLS - 1)), (0, 0))), biasmask


def _featqkv_body(x_ref, wc_ref, bmask_ref, wqkv_ref, bqkv_ref, out_ref):
    acts = jnp.maximum(
        jnp.dot(x_ref[...], wc_ref[...], preferred_element_type=jnp.float32)
        + bmask_ref[...], 0.0)
    bm = acts.shape[0]
    feats = jnp.max(acts.reshape(bm, len(_KS), _PMAX, _FN), axis=2)
    feats = feats.reshape(bm, _AFN)
    out_ref[...] = (
        jnp.dot(feats, wqkv_ref[...], preferred_element_type=jnp.float32)
        + bqkv_ref[...])


def _scores_body(q_ref, k_ref, out_ref):
    q = q_ref[0]
    k = k_ref[0]
    outs = []
    for h in range(_H):
        qh = q[:, h * _DH:(h + 1) * _DH]
        kh = k[:, h * _DH:(h + 1) * _DH]
        outs.append(
            jax.lax.dot_general(qh, kh, (((1,), (1,)), ((), ())),
                                preferred_element_type=jnp.float32)
            / math.sqrt(_DH))
    out_ref[0] = jnp.concatenate(outs, axis=1)


def _pooled_body(v_ref, wa_ref, ba_ref, wm_ref, bm_ref, out_ref):
    vmean = jnp.sum(v_ref[...], axis=1) / _N
    ctx = jnp.dot(vmean, wa_ref[...], preferred_element_type=jnp.float32) + ba_ref[...]
    logits = jnp.dot(ctx, wm_ref[...], preferred_element_type=jnp.float32) + bm_ref[...]
    padded = jnp.concatenate([logits, jnp.zeros((_B, 126), jnp.float32)], axis=1)
    out_ref[...] = jnp.concatenate(
        [padded, jnp.zeros((8 - _B, 128), jnp.float32)], axis=0)


_NROWS = _B * _N
_NW = 32
_RPW = _NROWS // _NW
_OPAD = 128


def _sorted_topk_vregs(ks, vs, desc):
    n = len(ks)
    if n == 1:
        k, v = plsc.sort_key_val(ks[0], vs[0], descending=desc)
        return [k], [v]
    ak, av = _sorted_topk_vregs(ks[:n // 2], vs[:n // 2], desc)
    bk, bv = _sorted_topk_vregs(ks[n // 2:], vs[n // 2:], not desc)
    return _bitonic_merge(ak + bk, av + bv, desc)


def _bitonic_merge(ks, vs, desc):
    n = len(ks)
    if n == 1:
        k, v = plsc.sort_key_val(ks[0], vs[0], descending=desc)
        return [k], [v]
    half = n // 2
    ks = list(ks)
    vs = list(vs)
    for i in range(half):
        a_k, b_k = ks[i], ks[i + half]
        a_v, b_v = vs[i], vs[i + half]
        m = (a_k >= b_k) if desc else (a_k <= b_k)
        ks[i] = jnp.where(m, a_k, b_k)
        ks[i + half] = jnp.where(m, b_k, a_k)
        vs[i] = jnp.where(m, a_v, b_v)
        vs[i + half] = jnp.where(m, b_v, a_v)
    k1, v1 = _bitonic_merge(ks[:half], vs[:half], desc)
    k2, v2 = _bitonic_merge(ks[half:], vs[half:], desc)
    return k1 + k2, v1 + v2


def _route_body(s4_hbm, out_hbm, s4row, keys, hist1k, cge1k, ckey, cidx,
                hist32, tkk, tki, eqb, outrow, sgbuf, dmasem):
    cid = lax.axis_index("c")
    sid = lax.axis_index("s")
    wid = sid * 2 + cid
    row0 = wid * _RPW
    iota = lax.broadcasted_iota(jnp.int32, (16,), 0)
    ones_i = jnp.ones((16,), jnp.int32)
    zeros_i = jnp.zeros((16,), jnp.int32)
    NV = _N // 16
    NB = 1024

    pltpu.async_copy(s4_hbm.at[row0], s4row, dmasem)

    def row_body(r, _c):
        row = row0 + r
        pltpu.make_async_copy(s4_hbm.at[row], s4row, dmasem).wait()

        for v in range(NB // 16):
            hist1k[pl.ds(v * 16, 16)] = zeros_i

        def pass_a(i, c):
            for j in range(4):
                o = i * 64 + j * 16
                s = (s4row[pl.ds(o, 16)] + s4row[pl.ds(_N + o, 16)]
                     + s4row[pl.ds(2 * _N + o, 16)]
                     + s4row[pl.ds(3 * _N + o, 16)])
                b = plsc.bitcast(s, jnp.uint32)
                neg = b >= jnp.uint32(0x80000000)
                key = jnp.where(neg, ~b, b | jnp.uint32(0x80000000))
                keys[pl.ds(o, 16)] = plsc.bitcast(key, jnp.int32)
                dig = (key >> 22).astype(jnp.int32)
                plsc.addupdate_scatter(hist1k, [dig], ones_i)
            return c
        lax.fori_loop(0, NV // 4, pass_a, 0)

        def suf(vi, carry):
            v = NB // 16 - 1 - vi
            h = hist1k[pl.ds(v * 16, 16)]
            s = lax.rev(plsc.cumsum(lax.rev(h, (0,))), (0,))
            cge1k[pl.ds(v * 16, 16)] = s + carry
            return carry + jnp.sum(h)
        lax.fori_loop(0, NB // 16, suf, 0)

        def cnt_ge(vi, acc):
            c = cge1k[pl.ds(vi * 16, 16)]
            return acc + (c >= _TOPK).astype(jnp.int32)
        g = jnp.sum(lax.fori_loop(0, NB // 16, cnt_ge, zeros_i)) - 1
        gp = jnp.minimum(g + 1, NB - 1)
        m_gt = jnp.where(
            g >= NB - 1, 0,
            plsc.load_gather(cge1k, [jnp.broadcast_to(gp, (16,))]))

        def pass_b(i, carry):
            off_hi, off_eq = carry
            for j in range(2):
                o = i * 32 + j * 16
                key = plsc.bitcast(keys[pl.ds(o, 16)], jnp.uint32)
                top = (key >> 22).astype(jnp.int32)
                idxv = o + iota
                m_hi = top > g
                m_eq = top == g
                pos_hi = plsc.cumsum(ones_i, mask=m_hi) - 1 + off_hi
                pos_hi = jnp.where(m_hi, pos_hi, 0)
                plsc.store_scatter(tkk, [pos_hi],
                                   plsc.bitcast(key, jnp.int32), mask=m_hi)
                plsc.store_scatter(tki, [pos_hi], idxv, mask=m_hi)
                pos_eq = plsc.cumsum(ones_i, mask=m_eq) - 1 + off_eq
                pos_eq = jnp.where(m_eq, pos_eq, 0)
                plsc.store_scatter(ckey, [pos_eq],
                                   plsc.bitcast(key, jnp.int32), mask=m_eq)
                plsc.store_scatter(cidx, [pos_eq], idxv, mask=m_eq)
                off_hi = off_hi + plsc.all_reduce_population_count(m_hi)
                off_eq = off_eq + plsc.all_reduce_population_count(m_eq)
            return (off_hi, off_eq)
        _, ccount_v = lax.fori_loop(0, NV // 2, pass_b, (zeros_i, zeros_i))
        ccount = jnp.max(ccount_v)
        ctiles = (ccount + 15) // 16
        need = _TOPK - m_gt
        prefix = jnp.broadcast_to(g.astype(jnp.uint32), (16,))

        for shift, width in ((17, 5), (12, 5), (7, 5), (2, 5), (0, 2)):
            nb = 1 << width
            topshift = shift + width
            hist32[pl.ds(0, 16)] = zeros_i
            hist32[pl.ds(16, 16)] = zeros_i

            def ph(t, c, shift=shift, topshift=topshift, nb=nb, prefix=prefix):
                key = plsc.bitcast(ckey[pl.ds(t * 16, 16)], jnp.uint32)
                valid = (t * 16 + iota) < ccount
                alive = jnp.logical_and((key >> topshift) == prefix, valid)
                dig = ((key >> shift).astype(jnp.int32)) & (nb - 1)
                plsc.addupdate_scatter(hist32, [dig], ones_i, mask=alive)
                return c
            lax.fori_loop(0, ctiles, ph, 0)
            h_lo = hist32[pl.ds(0, 16)]
            h_hi = hist32[pl.ds(16, 16)]
            s_hi = lax.rev(plsc.cumsum(lax.rev(h_hi, (0,))), (0,))
            s_lo = (lax.rev(plsc.cumsum(lax.rev(h_lo, (0,))), (0,))
                    + jnp.broadcast_to(jnp.sum(h_hi), (16,)))
            d = jnp.sum((s_lo >= need).astype(jnp.int32)
                        + (s_hi >= need).astype(jnp.int32)) - 1
            m_gt2 = jnp.sum(jnp.where(iota > d, h_lo, 0)
                            + jnp.where(iota + 16 > d, h_hi, 0))
            need = need - m_gt2
            prefix = (prefix << width) | jnp.broadcast_to(
                d.astype(jnp.uint32), (16,))
        t_key = prefix

        def sel(t, off):
            key = plsc.bitcast(ckey[pl.ds(t * 16, 16)], jnp.uint32)
            idxv = cidx[pl.ds(t * 16, 16)]
            valid = (t * 16 + iota) < ccount
            m3 = jnp.logical_and(key > t_key, valid)
            pos = plsc.cumsum(ones_i, mask=m3) - 1 + off
            pos = jnp.where(m3, pos, 0)
            plsc.store_scatter(tkk, [pos], plsc.bitcast(key, jnp.int32), mask=m3)
            plsc.store_scatter(tki, [pos], idxv, mask=m3)
            return off + plsc.all_reduce_population_count(m3)
        off2 = lax.fori_loop(0, ctiles, sel, m_gt)

        def eqf(t, off):
            key = plsc.bitcast(ckey[pl.ds(t * 16, 16)], jnp.uint32)
            idxv = cidx[pl.ds(t * 16, 16)]
            valid = (t * 16 + iota) < ccount
            m4 = jnp.logical_and(key == t_key, valid)
            pos = plsc.cumsum(ones_i, mask=m4) - 1 + off
            m4c = jnp.logical_and(m4, pos < 128)
            pos = jnp.where(m4c, pos, 0)
            plsc.store_scatter(eqb, [pos], idxv, mask=m4c)
            return off + plsc.all_reduce_population_count(m4)
        lax.fori_loop(0, ctiles, eqf, zeros_i)
        for j in range(7):
            e = eqb[pl.ds(j * 16, 16)]
            pos = off2 + j * 16 + iota
            m5 = pos < _TOPK
            posc = jnp.where(m5, pos, 0)
            plsc.store_scatter(tki, [posc], e, mask=m5)
            plsc.store_scatter(tkk, [posc], plsc.bitcast(t_key, jnp.int32), mask=m5)

        v6k = tkk[pl.ds(96, 16)]
        v6i = tki[pl.ds(96, 16)]
        mpad = (96 + iota) >= _TOPK
        tkk[pl.ds(96, 16)] = jnp.where(mpad, 0, v6k)
        tki[pl.ds(96, 16)] = jnp.where(mpad, 0, v6i)
        tkk[pl.ds(112, 16)] = zeros_i
        tki[pl.ds(112, 16)] = zeros_i

        kregs = [plsc.bitcast(tkk[pl.ds(j * 16, 16)], jnp.uint32)
                 for j in range(8)]
        iregs = [tki[pl.ds(j * 16, 16)] for j in range(8)]
        sk, si = _sorted_topk_vregs(kregs, iregs, True)

        for h in range(_H):
            for j in range(7):
                sgbuf[pl.ds(h * 112 + j * 16, 16)] = plsc.load_gather(
                    s4row, [si[j] + h * _N])
        nxt = jnp.minimum(row + 1, row0 + _RPW - 1)
        pltpu.async_copy(s4_hbm.at[nxt], s4row, dmasem)

        for h in range(_H):
            gs = [sgbuf[pl.ds(h * 112 + j * 16, 16)] for j in range(7)]
            lanes = [j * 16 + iota for j in range(7)]
            gm = [jnp.where(lanes[j] < _TOPK, gs[j], -1e30) for j in range(7)]
            mx = gm[0]
            for j in range(1, 7):
                mx = jnp.maximum(mx, gm[j])
            mxs = jnp.max(mx)
            es = [jnp.where(lanes[j] < _TOPK,
                            jnp.exp(gs[j] - mxs), 0.0) for j in range(7)]
            tot = es[0]
            for j in range(1, 7):
                tot = tot + es[j]
            ssum = jnp.sum(tot)
            for j in range(7):
                outrow[pl.ds(h * _OPAD + j * 16, 16)] = es[j] / ssum
            outrow[pl.ds(h * _OPAD + 112, 16)] = jnp.zeros((16,), jnp.float32)

        pltpu.sync_copy(outrow, out_hbm.at[row])
        return _c

    lax.fori_loop(0, _RPW, row_body, 0)
    pltpu.make_async_copy(
        s4_hbm.at[row0 + _RPW - 1], s4row, dmasem).wait()


def _make_route_kernel():
    mesh = plsc.VectorSubcoreMesh(core_axis_name="c", subcore_axis_name="s")
    return pl.kernel(
        _route_body,
        out_type=jax.ShapeDtypeStruct((_NROWS, _H * _OPAD), jnp.float32),
        mesh=mesh,
        compiler_params=pltpu.CompilerParams(needs_layout_passes=False),
        scratch_types=[
            pltpu.VMEM((_H * _N,), jnp.float32),
            pltpu.VMEM((_N,), jnp.int32),
            pltpu.VMEM((1024,), jnp.int32),
            pltpu.VMEM((1024,), jnp.int32),
            pltpu.VMEM((_N,), jnp.int32),
            pltpu.VMEM((_N,), jnp.int32),
            pltpu.VMEM((32,), jnp.int32),
            pltpu.VMEM((128,), jnp.int32),
            pltpu.VMEM((128,), jnp.int32),
            pltpu.VMEM((128,), jnp.int32),
            pltpu.VMEM((_H * _OPAD,), jnp.float32),
            pltpu.VMEM((_H * 112,), jnp.float32),
            pltpu.SemaphoreType.DMA,
        ],
    )


def kernel(x, cw2, cb2, cw3, cb3, cw4, cb4, cw5, cb5, cw6, cb6, cw7, cb7,
           Wq, bq, Wk, bk, Wv, bv, Wa, ba, Wm, bm):
    NR = _B * _N
    xf = x.reshape(NR, _FEAT * _AA)
    xa = jnp.concatenate(
        [xf, jnp.zeros((NR, _XPAD - (_XCOLS - 1)), jnp.float32)], axis=1)
    Wc, biasmask = _build_conv_matrix((cw2, cw3, cw4, cw5, cw6, cw7),
                                      (cb2, cb3, cb4, cb5, cb6, cb7))
    Wqkv = jnp.concatenate([Wq.T, Wk.T, Wv.T], axis=1)
    bqkv = jnp.concatenate([bq, bk, bv])[None, :]

    BM = 512
    qkv = pl.pallas_call(
        _featqkv_body,
        grid=(NR // BM,),
        in_specs=[
            pl.BlockSpec((BM, _XPAD), lambda i: (i, 0)),
            pl.BlockSpec((_XPAD, _CCOLS), lambda i: (0, 0)),
            pl.BlockSpec((1, _CCOLS), lambda i: (0, 0)),
            pl.BlockSpec((_AFN, 3 * _HID), lambda i: (0, 0)),
            pl.BlockSpec((1, 3 * _HID), lambda i: (0, 0)),
        ],
        out_specs=pl.BlockSpec((BM, 3 * _HID), lambda i: (i, 0)),
        out_shape=jax.ShapeDtypeStruct((NR, 3 * _HID), jnp.float32),
    )(xa, Wc, biasmask, Wqkv, bqkv)

    q = qkv[:, :_HID].reshape(_B, _N, _HID)
    k = qkv[:, _HID:2 * _HID].reshape(_B, _N, _HID)
    v = qkv[:, 2 * _HID:].reshape(_B, _N, _HID)

    BS = 256
    s4 = pl.pallas_call(
        _scores_body,
        grid=(_B, _N // BS),
        in_specs=[
            pl.BlockSpec((1, BS, _HID), lambda b, i: (b, i, 0)),
            pl.BlockSpec((1, _N, _HID), lambda b, i: (b, 0, 0)),
        ],
        out_specs=pl.BlockSpec((1, BS, _H * _N), lambda b, i: (b, i, 0)),
        out_shape=jax.ShapeDtypeStruct((_B, _N, _H * _N), jnp.float32),
    )(q, k)

    probs_pad = _make_route_kernel()(s4.reshape(_NROWS, _H * _N))
    probs = probs_pad.reshape(_B, _N, _H, _OPAD)[:, :, :, :_TOPK]
    probs_e = probs.transpose(0, 2, 1, 3)[:, :, :, None, :]

    pooled_pad = pl.pallas_call(
        _pooled_body,
        in_specs=[
            pl.BlockSpec((_B, _N, _HID), lambda: (0, 0, 0)),
            pl.BlockSpec((_HID, _AFN), lambda: (0, 0)),
            pl.BlockSpec((1, _AFN), lambda: (0, 0)),
            pl.BlockSpec((_AFN, 2), lambda: (0, 0)),
            pl.BlockSpec((1, 2), lambda: (0, 0)),
        ],
        out_specs=pl.BlockSpec((8, 128), lambda: (0, 0)),
        out_shape=jax.ShapeDtypeStruct((8, 128), jnp.float32),
    )(v, Wa.T, ba[None, :], Wm.T, bm[None, :])
    pooled = pooled_pad[:_B, :2]

    return pooled, probs_e


# speedup vs baseline: 1.3619x; 1.0165x over previous
import functools
import math

import jax
import jax.numpy as jnp
from jax import lax
from jax.experimental import pallas as pl
from jax.experimental.pallas import tpu as pltpu
from jax.experimental.pallas import tpu_sc as plsc

_B = 2
_N = 2048
_FEAT = 15
_AA = 24
_KS = (2, 3, 4, 5, 6, 7)
_FN = 32
_AFN = _FN * len(_KS)
_H = 4
_HID = 128
_DH = 32
_TOPK = 102
_PMAX = _AA
_CCOLS = len(_KS) * _PMAX * _FN
_XCOLS = _FEAT * _AA + 1
_XPAD = 384


def _build_conv_matrix(cws, cbs):
    blocks = []
    bias_cols = []
    for g, h in enumerate(_KS):
        w = cws[g].transpose(1, 2, 0)
        cols = []
        np_valid = _AA - h + 1
        for p in range(_PMAX):
            if p < np_valid:
                m = jnp.zeros((_FEAT, _AA, _FN), jnp.float32)
                m = lax.dynamic_update_slice(m, w, (0, p, 0))
                m = m.reshape(_FEAT * _AA, _FN)
                bias_cols.append(cbs[g])
            else:
                m = jnp.zeros((_FEAT * _AA, _FN), jnp.float32)
                bias_cols.append(jnp.full((_FN,), -1e30))
            cols.append(m)
        blocks.append(jnp.concatenate(cols, axis=1))
    W = jnp.concatenate(blocks, axis=1)
    biasmask = jnp.concatenate(bias_cols)[None, :]
    return jnp.pad(W, ((0, _XPAD - (_XCOLS - 1)), (0, 0))), biasmask


def _featqkv_body(x_ref, wc_ref, bmask_ref, wqkv_ref, bqkv_ref, out_ref):
    acts = jnp.maximum(
        jnp.dot(x_ref[...], wc_ref[...], preferred_element_type=jnp.float32)
        + bmask_ref[...], 0.0)
    bm = acts.shape[0]
    feats = jnp.max(acts.reshape(bm, len(_KS), _PMAX, _FN), axis=2)
    feats = feats.reshape(bm, _AFN)
    out_ref[...] = (
        jnp.dot(feats, wqkv_ref[...], preferred_element_type=jnp.float32)
        + bqkv_ref[...])


def _scores_body(q_ref, k_ref, out_ref):
    q = q_ref[0]
    k = k_ref[0]
    outs = []
    for h in range(_H):
        qh = q[:, h * _DH:(h + 1) * _DH]
        kh = k[:, h * _DH:(h + 1) * _DH]
        outs.append(
            jax.lax.dot_general(qh, kh, (((1,), (1,)), ((), ())),
                                preferred_element_type=jnp.float32)
            / math.sqrt(_DH))
    out_ref[0] = jnp.concatenate(outs, axis=1)


def _pooled_body(v_ref, wa_ref, ba_ref, wm_ref, bm_ref, out_ref):
    vmean = jnp.sum(v_ref[...], axis=1) / _N
    ctx = jnp.dot(vmean, wa_ref[...], preferred_element_type=jnp.float32) + ba_ref[...]
    logits = jnp.dot(ctx, wm_ref[...], preferred_element_type=jnp.float32) + bm_ref[...]
    padded = jnp.concatenate([logits, jnp.zeros((_B, 126), jnp.float32)], axis=1)
    out_ref[...] = jnp.concatenate(
        [padded, jnp.zeros((8 - _B, 128), jnp.float32)], axis=0)


_NROWS = _B * _N
_NW = 32
_RPW = _NROWS // _NW
_OPAD = 128


def _sorted_topk_vregs(ks, vs, desc):
    n = len(ks)
    if n == 1:
        k, v = plsc.sort_key_val(ks[0], vs[0], descending=desc)
        return [k], [v]
    ak, av = _sorted_topk_vregs(ks[:n // 2], vs[:n // 2], desc)
    bk, bv = _sorted_topk_vregs(ks[n // 2:], vs[n // 2:], not desc)
    return _bitonic_merge(ak + bk, av + bv, desc)


def _bitonic_merge(ks, vs, desc):
    n = len(ks)
    if n == 1:
        k, v = plsc.sort_key_val(ks[0], vs[0], descending=desc)
        return [k], [v]
    half = n // 2
    ks = list(ks)
    vs = list(vs)
    for i in range(half):
        a_k, b_k = ks[i], ks[i + half]
        a_v, b_v = vs[i], vs[i + half]
        m = (a_k >= b_k) if desc else (a_k <= b_k)
        ks[i] = jnp.where(m, a_k, b_k)
        ks[i + half] = jnp.where(m, b_k, a_k)
        vs[i] = jnp.where(m, a_v, b_v)
        vs[i + half] = jnp.where(m, b_v, a_v)
    k1, v1 = _bitonic_merge(ks[:half], vs[:half], desc)
    k2, v2 = _bitonic_merge(ks[half:], vs[half:], desc)
    return k1 + k2, v1 + v2


def _route_body(s4_hbm, out_hbm, s4row, keys, hist1k, cge1k, ckey, cidx,
                hist32, tkk, tki, eqb, outrow, sgbuf, dmasem):
    cid = lax.axis_index("c")
    sid = lax.axis_index("s")
    wid = sid * 2 + cid
    row0 = wid * _RPW
    iota = lax.broadcasted_iota(jnp.int32, (16,), 0)
    ones_i = jnp.ones((16,), jnp.int32)
    zeros_i = jnp.zeros((16,), jnp.int32)
    NV = _N // 16
    NB = 1024

    pltpu.async_copy(s4_hbm.at[row0], s4row, dmasem)

    def row_body(r, _c):
        row = row0 + r
        pltpu.make_async_copy(s4_hbm.at[row], s4row, dmasem).wait()

        for v in range(NB // 16):
            hist1k[pl.ds(v * 16, 16)] = zeros_i

        def pass_a(i, c):
            for j in range(4):
                o = i * 64 + j * 16
                s = (s4row[pl.ds(o, 16)] + s4row[pl.ds(_N + o, 16)]
                     + s4row[pl.ds(2 * _N + o, 16)]
                     + s4row[pl.ds(3 * _N + o, 16)])
                b = plsc.bitcast(s, jnp.uint32)
                neg = b >= jnp.uint32(0x80000000)
                key = jnp.where(neg, ~b, b | jnp.uint32(0x80000000))
                keys[pl.ds(o, 16)] = plsc.bitcast(key, jnp.int32)
                dig = (key >> 22).astype(jnp.int32)
                plsc.addupdate_scatter(hist1k, [dig], ones_i)
            return c
        lax.fori_loop(0, NV // 4, pass_a, 0)

        def suf(vi, carry):
            cacc, cnt = carry
            v = NB // 16 - 1 - vi
            h = hist1k[pl.ds(v * 16, 16)]
            s = lax.rev(plsc.cumsum(lax.rev(h, (0,))), (0,)) + cacc
            cge1k[pl.ds(v * 16, 16)] = s
            return (cacc + jnp.sum(h), cnt + (s >= _TOPK).astype(jnp.int32))
        _, gcnt = lax.fori_loop(0, NB // 16, suf, (0, zeros_i))
        g = jnp.sum(gcnt) - 1
        gp = jnp.minimum(g + 1, NB - 1)
        m_gt = jnp.where(
            g >= NB - 1, 0,
            plsc.load_gather(cge1k, [jnp.broadcast_to(gp, (16,))]))

        def pass_b(i, off_eq):
            for j in range(2):
                o = i * 32 + j * 16
                key = plsc.bitcast(keys[pl.ds(o, 16)], jnp.uint32)
                m_eq = (key >> 22).astype(jnp.int32) >= g
                pos_eq = plsc.cumsum(ones_i, mask=m_eq) - 1 + off_eq
                pos_eq = jnp.where(m_eq, pos_eq, 0)
                plsc.store_scatter(ckey, [pos_eq],
                                   plsc.bitcast(key, jnp.int32), mask=m_eq)
                plsc.store_scatter(cidx, [pos_eq], o + iota, mask=m_eq)
                off_eq = off_eq + plsc.all_reduce_population_count(m_eq)
            return off_eq
        ccount_v = lax.fori_loop(0, NV // 2, pass_b, zeros_i)
        ccount = jnp.max(ccount_v)
        ctiles = (ccount + 15) // 16
        need = _TOPK - m_gt
        prefix = jnp.broadcast_to(g.astype(jnp.uint32), (16,))

        for shift, width in ((17, 5), (12, 5), (7, 5), (2, 5), (0, 2)):
            nb = 1 << width
            topshift = shift + width
            hist32[pl.ds(0, 16)] = zeros_i
            hist32[pl.ds(16, 16)] = zeros_i

            def ph(t, c, shift=shift, topshift=topshift, nb=nb, prefix=prefix):
                key = plsc.bitcast(ckey[pl.ds(t * 16, 16)], jnp.uint32)
                valid = (t * 16 + iota) < ccount
                alive = jnp.logical_and((key >> topshift) == prefix, valid)
                dig = ((key >> shift).astype(jnp.int32)) & (nb - 1)
                plsc.addupdate_scatter(hist32, [dig], ones_i, mask=alive)
                return c
            lax.fori_loop(0, ctiles, ph, 0)
            h_lo = hist32[pl.ds(0, 16)]
            h_hi = hist32[pl.ds(16, 16)]
            s_hi = lax.rev(plsc.cumsum(lax.rev(h_hi, (0,))), (0,))
            s_lo = (lax.rev(plsc.cumsum(lax.rev(h_lo, (0,))), (0,))
                    + jnp.broadcast_to(jnp.sum(h_hi), (16,)))
            d = jnp.sum((s_lo >= need).astype(jnp.int32)
                        + (s_hi >= need).astype(jnp.int32)) - 1
            m_gt2 = jnp.sum(jnp.where(iota > d, h_lo, 0)
                            + jnp.where(iota + 16 > d, h_hi, 0))
            need = need - m_gt2
            prefix = (prefix << width) | jnp.broadcast_to(
                d.astype(jnp.uint32), (16,))
        t_key = prefix

        def sel(t, off):
            key = plsc.bitcast(ckey[pl.ds(t * 16, 16)], jnp.uint32)
            idxv = cidx[pl.ds(t * 16, 16)]
            valid = (t * 16 + iota) < ccount
            m3 = jnp.logical_and(key > t_key, valid)
            pos = plsc.cumsum(ones_i, mask=m3) - 1 + off
            pos = jnp.where(m3, pos, 0)
            plsc.store_scatter(tkk, [pos], plsc.bitcast(key, jnp.int32), mask=m3)
            plsc.store_scatter(tki, [pos], idxv, mask=m3)
            return off + plsc.all_reduce_population_count(m3)
        off2 = lax.fori_loop(0, ctiles, sel, zeros_i)

        def eqf(t, off):
            key = plsc.bitcast(ckey[pl.ds(t * 16, 16)], jnp.uint32)
            idxv = cidx[pl.ds(t * 16, 16)]
            valid = (t * 16 + iota) < ccount
            m4 = jnp.logical_and(key == t_key, valid)
            pos = plsc.cumsum(ones_i, mask=m4) - 1 + off
            m4c = jnp.logical_and(m4, pos < 128)
            pos = jnp.where(m4c, pos, 0)
            plsc.store_scatter(eqb, [pos], idxv, mask=m4c)
            return off + plsc.all_reduce_population_count(m4)
        lax.fori_loop(0, ctiles, eqf, zeros_i)
        for j in range(7):
            e = eqb[pl.ds(j * 16, 16)]
            pos = off2 + j * 16 + iota
            m5 = pos < _TOPK
            posc = jnp.where(m5, pos, 0)
            plsc.store_scatter(tki, [posc], e, mask=m5)
            plsc.store_scatter(tkk, [posc], plsc.bitcast(t_key, jnp.int32), mask=m5)

        v6k = tkk[pl.ds(96, 16)]
        v6i = tki[pl.ds(96, 16)]
        mpad = (96 + iota) >= _TOPK
        tkk[pl.ds(96, 16)] = jnp.where(mpad, 0, v6k)
        tki[pl.ds(96, 16)] = jnp.where(mpad, 0, v6i)
        tkk[pl.ds(112, 16)] = zeros_i
        tki[pl.ds(112, 16)] = zeros_i

        kregs = [plsc.bitcast(tkk[pl.ds(j * 16, 16)], jnp.uint32)
                 for j in range(8)]
        iregs = [tki[pl.ds(j * 16, 16)] for j in range(8)]
        sk, si = _sorted_topk_vregs(kregs, iregs, True)

        for h in range(_H):
            for j in range(7):
                sgbuf[pl.ds(h * 112 + j * 16, 16)] = plsc.load_gather(
                    s4row, [si[j] + h * _N])
        nxt = jnp.minimum(row + 1, row0 + _RPW - 1)
        pltpu.async_copy(s4_hbm.at[nxt], s4row, dmasem)

        for h in range(_H):
            gs = [sgbuf[pl.ds(h * 112 + j * 16, 16)] for j in range(7)]
            lanes = [j * 16 + iota for j in range(7)]
            gm = [jnp.where(lanes[j] < _TOPK, gs[j], -1e30) for j in range(7)]
            mx = gm[0]
            for j in range(1, 7):
                mx = jnp.maximum(mx, gm[j])
            mxs = jnp.max(mx)
            es = [jnp.where(lanes[j] < _TOPK,
                            jnp.exp(gs[j] - mxs), 0.0) for j in range(7)]
            tot = es[0]
            for j in range(1, 7):
                tot = tot + es[j]
            ssum = jnp.sum(tot)
            for j in range(7):
                outrow[pl.ds(h * _OPAD + j * 16, 16)] = es[j] / ssum
            outrow[pl.ds(h * _OPAD + 112, 16)] = jnp.zeros((16,), jnp.float32)

        pltpu.sync_copy(outrow, out_hbm.at[row])
        return _c

    lax.fori_loop(0, _RPW, row_body, 0)
    pltpu.make_async_copy(
        s4_hbm.at[row0 + _RPW - 1], s4row, dmasem).wait()


def _make_route_kernel():
    mesh = plsc.VectorSubcoreMesh(core_axis_name="c", subcore_axis_name="s")
    return pl.kernel(
        _route_body,
        out_type=jax.ShapeDtypeStruct((_NROWS, _H * _OPAD), jnp.float32),
        mesh=mesh,
        compiler_params=pltpu.CompilerParams(needs_layout_passes=False),
        scratch_types=[
            pltpu.VMEM((_H * _N,), jnp.float32),
            pltpu.VMEM((_N,), jnp.int32),
            pltpu.VMEM((1024,), jnp.int32),
            pltpu.VMEM((1024,), jnp.int32),
            pltpu.VMEM((_N,), jnp.int32),
            pltpu.VMEM((_N,), jnp.int32),
            pltpu.VMEM((32,), jnp.int32),
            pltpu.VMEM((128,), jnp.int32),
            pltpu.VMEM((128,), jnp.int32),
            pltpu.VMEM((128,), jnp.int32),
            pltpu.VMEM((_H * _OPAD,), jnp.float32),
            pltpu.VMEM((_H * 112,), jnp.float32),
            pltpu.SemaphoreType.DMA,
        ],
    )


def kernel(x, cw2, cb2, cw3, cb3, cw4, cb4, cw5, cb5, cw6, cb6, cw7, cb7,
           Wq, bq, Wk, bk, Wv, bv, Wa, ba, Wm, bm):
    NR = _B * _N
    xf = x.reshape(NR, _FEAT * _AA)
    xa = jnp.concatenate(
        [xf, jnp.zeros((NR, _XPAD - (_XCOLS - 1)), jnp.float32)], axis=1)
    Wc, biasmask = _build_conv_matrix((cw2, cw3, cw4, cw5, cw6, cw7),
                                      (cb2, cb3, cb4, cb5, cb6, cb7))
    Wqkv = jnp.concatenate([Wq.T, Wk.T, Wv.T], axis=1)
    bqkv = jnp.concatenate([bq, bk, bv])[None, :]

    BM = 512
    qkv = pl.pallas_call(
        _featqkv_body,
        grid=(NR // BM,),
        in_specs=[
            pl.BlockSpec((BM, _XPAD), lambda i: (i, 0)),
            pl.BlockSpec((_XPAD, _CCOLS), lambda i: (0, 0)),
            pl.BlockSpec((1, _CCOLS), lambda i: (0, 0)),
            pl.BlockSpec((_AFN, 3 * _HID), lambda i: (0, 0)),
            pl.BlockSpec((1, 3 * _HID), lambda i: (0, 0)),
        ],
        out_specs=pl.BlockSpec((BM, 3 * _HID), lambda i: (i, 0)),
        out_shape=jax.ShapeDtypeStruct((NR, 3 * _HID), jnp.float32),
    )(xa, Wc, biasmask, Wqkv, bqkv)

    q = qkv[:, :_HID].reshape(_B, _N, _HID)
    k = qkv[:, _HID:2 * _HID].reshape(_B, _N, _HID)
    v = qkv[:, 2 * _HID:].reshape(_B, _N, _HID)

    BS = 256
    s4 = pl.pallas_call(
        _scores_body,
        grid=(_B, _N // BS),
        in_specs=[
            pl.BlockSpec((1, BS, _HID), lambda b, i: (b, i, 0)),
            pl.BlockSpec((1, _N, _HID), lambda b, i: (b, 0, 0)),
        ],
        out_specs=pl.BlockSpec((1, BS, _H * _N), lambda b, i: (b, i, 0)),
        out_shape=jax.ShapeDtypeStruct((_B, _N, _H * _N), jnp.float32),
    )(q, k)

    probs_pad = _make_route_kernel()(s4.reshape(_NROWS, _H * _N))
    probs = probs_pad.reshape(_B, _N, _H, _OPAD)[:, :, :, :_TOPK]
    probs_e = probs.transpose(0, 2, 1, 3)[:, :, :, None, :]

    pooled_pad = pl.pallas_call(
        _pooled_body,
        in_specs=[
            pl.BlockSpec((_B, _N, _HID), lambda: (0, 0, 0)),
            pl.BlockSpec((_HID, _AFN), lambda: (0, 0)),
            pl.BlockSpec((1, _AFN), lambda: (0, 0)),
            pl.BlockSpec((_AFN, 2), lambda: (0, 0)),
            pl.BlockSpec((1, 2), lambda: (0, 0)),
        ],
        out_specs=pl.BlockSpec((8, 128), lambda: (0, 0)),
        out_shape=jax.ShapeDtypeStruct((8, 128), jnp.float32),
    )(v, Wa.T, ba[None, :], Wm.T, bm[None, :])
    pooled = pooled_pad[:_B, :2]

    return pooled, probs_e


# speedup vs baseline: 1.3658x; 1.0028x over previous
import functools
import math

import jax
import jax.numpy as jnp
from jax import lax
from jax.experimental import pallas as pl
from jax.experimental.pallas import tpu as pltpu
from jax.experimental.pallas import tpu_sc as plsc

_B = 2
_N = 2048
_FEAT = 15
_AA = 24
_KS = (2, 3, 4, 5, 6, 7)
_FN = 32
_AFN = _FN * len(_KS)
_H = 4
_HID = 128
_DH = 32
_TOPK = 102
_PMAX = _AA
_CCOLS = len(_KS) * _PMAX * _FN
_XCOLS = _FEAT * _AA + 1
_XPAD = 384


def _build_conv_matrix(cws, cbs):
    blocks = []
    bias_cols = []
    for g, h in enumerate(_KS):
        w = cws[g].transpose(1, 2, 0)
        cols = []
        np_valid = _AA - h + 1
        for p in range(_PMAX):
            if p < np_valid:
                m = jnp.zeros((_FEAT, _AA, _FN), jnp.float32)
                m = lax.dynamic_update_slice(m, w, (0, p, 0))
                m = m.reshape(_FEAT * _AA, _FN)
                bias_cols.append(cbs[g])
            else:
                m = jnp.zeros((_FEAT * _AA, _FN), jnp.float32)
                bias_cols.append(jnp.full((_FN,), -1e30))
            cols.append(m)
        blocks.append(jnp.concatenate(cols, axis=1))
    W = jnp.concatenate(blocks, axis=1)
    biasmask = jnp.concatenate(bias_cols)[None, :]
    return jnp.pad(W, ((0, _XPAD - (_XCOLS - 1)), (0, 0))), biasmask


def _featqkv_body(x_ref, wc_ref, bmask_ref, wqkv_ref, bqkv_ref, out_ref):
    acts = jnp.maximum(
        jnp.dot(x_ref[...], wc_ref[...], preferred_element_type=jnp.float32)
        + bmask_ref[...], 0.0)
    bm = acts.shape[0]
    feats = jnp.max(acts.reshape(bm, len(_KS), _PMAX, _FN), axis=2)
    feats = feats.reshape(bm, _AFN)
    out_ref[...] = (
        jnp.dot(feats, wqkv_ref[...], preferred_element_type=jnp.float32)
        + bqkv_ref[...])


def _scores_body(q_ref, k_ref, out_ref):
    q = q_ref[0]
    k = k_ref[0]
    outs = []
    for h in range(_H):
        qh = q[:, h * _DH:(h + 1) * _DH]
        kh = k[:, h * _DH:(h + 1) * _DH]
        outs.append(
            jax.lax.dot_general(qh, kh, (((1,), (1,)), ((), ())),
                                preferred_element_type=jnp.float32))
    out_ref[0] = jnp.concatenate(outs, axis=1)


def _pooled_body(v_ref, wa_ref, ba_ref, wm_ref, bm_ref, out_ref):
    vmean = jnp.sum(v_ref[...], axis=1) / _N
    ctx = jnp.dot(vmean, wa_ref[...], preferred_element_type=jnp.float32) + ba_ref[...]
    logits = jnp.dot(ctx, wm_ref[...], preferred_element_type=jnp.float32) + bm_ref[...]
    padded = jnp.concatenate([logits, jnp.zeros((_B, 126), jnp.float32)], axis=1)
    out_ref[...] = jnp.concatenate(
        [padded, jnp.zeros((8 - _B, 128), jnp.float32)], axis=0)


_NROWS = _B * _N
_NW = 32
_RPW = _NROWS // _NW
_OPAD = 128


def _sorted_topk_vregs(ks, vs, desc):
    n = len(ks)
    if n == 1:
        k, v = plsc.sort_key_val(ks[0], vs[0], descending=desc)
        return [k], [v]
    ak, av = _sorted_topk_vregs(ks[:n // 2], vs[:n // 2], desc)
    bk, bv = _sorted_topk_vregs(ks[n // 2:], vs[n // 2:], not desc)
    return _bitonic_merge(ak + bk, av + bv, desc)


def _bitonic_merge(ks, vs, desc):
    n = len(ks)
    if n == 1:
        k, v = plsc.sort_key_val(ks[0], vs[0], descending=desc)
        return [k], [v]
    half = n // 2
    ks = list(ks)
    vs = list(vs)
    for i in range(half):
        a_k, b_k = ks[i], ks[i + half]
        a_v, b_v = vs[i], vs[i + half]
        m = (a_k >= b_k) if desc else (a_k <= b_k)
        ks[i] = jnp.where(m, a_k, b_k)
        ks[i + half] = jnp.where(m, b_k, a_k)
        vs[i] = jnp.where(m, a_v, b_v)
        vs[i + half] = jnp.where(m, b_v, a_v)
    k1, v1 = _bitonic_merge(ks[:half], vs[:half], desc)
    k2, v2 = _bitonic_merge(ks[half:], vs[half:], desc)
    return k1 + k2, v1 + v2


def _route_body(s4_hbm, out_hbm, s4row, keys, hist1k, cge1k, ckey, cidx,
                hist32, tkk, tki, eqb, outrow, sgbuf, dmasem):
    cid = lax.axis_index("c")
    sid = lax.axis_index("s")
    wid = sid * 2 + cid
    row0 = wid * _RPW
    iota = lax.broadcasted_iota(jnp.int32, (16,), 0)
    ones_i = jnp.ones((16,), jnp.int32)
    zeros_i = jnp.zeros((16,), jnp.int32)
    NV = _N // 16
    NB = 1024

    pltpu.async_copy(s4_hbm.at[row0], s4row, dmasem)

    def row_body(r, _c):
        row = row0 + r
        pltpu.make_async_copy(s4_hbm.at[row], s4row, dmasem).wait()

        for v in range(NB // 16):
            hist1k[pl.ds(v * 16, 16)] = zeros_i

        def pass_a(i, c):
            for j in range(4):
                o = i * 64 + j * 16
                s = (s4row[pl.ds(o, 16)] + s4row[pl.ds(_N + o, 16)]
                     + s4row[pl.ds(2 * _N + o, 16)]
                     + s4row[pl.ds(3 * _N + o, 16)])
                b = plsc.bitcast(s, jnp.uint32)
                neg = b >= jnp.uint32(0x80000000)
                key = jnp.where(neg, ~b, b | jnp.uint32(0x80000000))
                keys[pl.ds(o, 16)] = plsc.bitcast(key, jnp.int32)
                dig = (key >> 22).astype(jnp.int32)
                plsc.addupdate_scatter(hist1k, [dig], ones_i)
            return c
        lax.fori_loop(0, NV // 4, pass_a, 0)

        def suf(vi, carry):
            cacc, cnt = carry
            v = NB // 16 - 1 - vi
            h = hist1k[pl.ds(v * 16, 16)]
            s = lax.rev(plsc.cumsum(lax.rev(h, (0,))), (0,)) + cacc
            cge1k[pl.ds(v * 16, 16)] = s
            return (cacc + jnp.sum(h), cnt + (s >= _TOPK).astype(jnp.int32))
        _, gcnt = lax.fori_loop(0, NB // 16, suf, (0, zeros_i))
        g = jnp.sum(gcnt) - 1
        gp = jnp.minimum(g + 1, NB - 1)
        m_gt = jnp.where(
            g >= NB - 1, 0,
            plsc.load_gather(cge1k, [jnp.broadcast_to(gp, (16,))]))

        def pass_b(i, off_eq):
            for j in range(2):
                o = i * 32 + j * 16
                key = plsc.bitcast(keys[pl.ds(o, 16)], jnp.uint32)
                m_eq = (key >> 22).astype(jnp.int32) >= g
                pos_eq = plsc.cumsum(ones_i, mask=m_eq) - 1 + off_eq
                pos_eq = jnp.where(m_eq, pos_eq, 0)
                plsc.store_scatter(ckey, [pos_eq],
                                   plsc.bitcast(key, jnp.int32), mask=m_eq)
                plsc.store_scatter(cidx, [pos_eq], o + iota, mask=m_eq)
                off_eq = off_eq + plsc.all_reduce_population_count(m_eq)
            return off_eq
        ccount_v = lax.fori_loop(0, NV // 2, pass_b, zeros_i)
        ccount = jnp.max(ccount_v)
        ctiles = (ccount + 15) // 16
        need = _TOPK - m_gt
        prefix = jnp.broadcast_to(g.astype(jnp.uint32), (16,))

        for shift, width in ((17, 5), (12, 5), (7, 5), (2, 5), (0, 2)):
            nb = 1 << width
            topshift = shift + width
            hist32[pl.ds(0, 16)] = zeros_i
            hist32[pl.ds(16, 16)] = zeros_i

            def ph(t, c, shift=shift, topshift=topshift, nb=nb, prefix=prefix):
                key = plsc.bitcast(ckey[pl.ds(t * 16, 16)], jnp.uint32)
                valid = (t * 16 + iota) < ccount
                alive = jnp.logical_and((key >> topshift) == prefix, valid)
                dig = ((key >> shift).astype(jnp.int32)) & (nb - 1)
                plsc.addupdate_scatter(hist32, [dig], ones_i, mask=alive)
                return c
            lax.fori_loop(0, ctiles, ph, 0)
            h_lo = hist32[pl.ds(0, 16)]
            h_hi = hist32[pl.ds(16, 16)]
            s_hi = lax.rev(plsc.cumsum(lax.rev(h_hi, (0,))), (0,))
            s_lo = (lax.rev(plsc.cumsum(lax.rev(h_lo, (0,))), (0,))
                    + jnp.broadcast_to(jnp.sum(h_hi), (16,)))
            d = jnp.sum((s_lo >= need).astype(jnp.int32)
                        + (s_hi >= need).astype(jnp.int32)) - 1
            m_gt2 = jnp.sum(jnp.where(iota > d, h_lo, 0)
                            + jnp.where(iota + 16 > d, h_hi, 0))
            need = need - m_gt2
            prefix = (prefix << width) | jnp.broadcast_to(
                d.astype(jnp.uint32), (16,))
        t_key = prefix

        def sel(t, off):
            key = plsc.bitcast(ckey[pl.ds(t * 16, 16)], jnp.uint32)
            idxv = cidx[pl.ds(t * 16, 16)]
            valid = (t * 16 + iota) < ccount
            m3 = jnp.logical_and(key > t_key, valid)
            pos = plsc.cumsum(ones_i, mask=m3) - 1 + off
            pos = jnp.where(m3, pos, 0)
            plsc.store_scatter(tkk, [pos], plsc.bitcast(key, jnp.int32), mask=m3)
            plsc.store_scatter(tki, [pos], idxv, mask=m3)
            return off + plsc.all_reduce_population_count(m3)
        off2 = lax.fori_loop(0, ctiles, sel, zeros_i)

        def eqf(t, off):
            key = plsc.bitcast(ckey[pl.ds(t * 16, 16)], jnp.uint32)
            idxv = cidx[pl.ds(t * 16, 16)]
            valid = (t * 16 + iota) < ccount
            m4 = jnp.logical_and(key == t_key, valid)
            pos = plsc.cumsum(ones_i, mask=m4) - 1 + off
            m4c = jnp.logical_and(m4, pos < 128)
            pos = jnp.where(m4c, pos, 0)
            plsc.store_scatter(eqb, [pos], idxv, mask=m4c)
            return off + plsc.all_reduce_population_count(m4)
        lax.fori_loop(0, ctiles, eqf, zeros_i)
        for j in range(7):
            e = eqb[pl.ds(j * 16, 16)]
            pos = off2 + j * 16 + iota
            m5 = pos < _TOPK
            posc = jnp.where(m5, pos, 0)
            plsc.store_scatter(tki, [posc], e, mask=m5)
            plsc.store_scatter(tkk, [posc], plsc.bitcast(t_key, jnp.int32), mask=m5)

        v6k = tkk[pl.ds(96, 16)]
        v6i = tki[pl.ds(96, 16)]
        mpad = (96 + iota) >= _TOPK
        tkk[pl.ds(96, 16)] = jnp.where(mpad, 0, v6k)
        tki[pl.ds(96, 16)] = jnp.where(mpad, 0, v6i)
        tkk[pl.ds(112, 16)] = zeros_i
        tki[pl.ds(112, 16)] = zeros_i

        kregs = [plsc.bitcast(tkk[pl.ds(j * 16, 16)], jnp.uint32)
                 for j in range(8)]
        iregs = [tki[pl.ds(j * 16, 16)] for j in range(8)]
        sk, si = _sorted_topk_vregs(kregs, iregs, True)

        for h in range(_H):
            for j in range(7):
                sgbuf[pl.ds(h * 112 + j * 16, 16)] = plsc.load_gather(
                    s4row, [si[j] + h * _N]) / math.sqrt(_DH)
        nxt = jnp.minimum(row + 1, row0 + _RPW - 1)
        pltpu.async_copy(s4_hbm.at[nxt], s4row, dmasem)

        for h in range(_H):
            gs = [sgbuf[pl.ds(h * 112 + j * 16, 16)] for j in range(7)]
            lanes = [j * 16 + iota for j in range(7)]
            gm = [jnp.where(lanes[j] < _TOPK, gs[j], -1e30) for j in range(7)]
            mx = gm[0]
            for j in range(1, 7):
                mx = jnp.maximum(mx, gm[j])
            mxs = jnp.max(mx)
            es = [jnp.where(lanes[j] < _TOPK,
                            jnp.exp(gs[j] - mxs), 0.0) for j in range(7)]
            tot = es[0]
            for j in range(1, 7):
                tot = tot + es[j]
            ssum = jnp.sum(tot)
            for j in range(7):
                outrow[pl.ds(h * _OPAD + j * 16, 16)] = es[j] / ssum
            outrow[pl.ds(h * _OPAD + 112, 16)] = jnp.zeros((16,), jnp.float32)

        pltpu.sync_copy(outrow, out_hbm.at[row])
        return _c

    lax.fori_loop(0, _RPW, row_body, 0)
    pltpu.make_async_copy(
        s4_hbm.at[row0 + _RPW - 1], s4row, dmasem).wait()


def _make_route_kernel():
    mesh = plsc.VectorSubcoreMesh(core_axis_name="c", subcore_axis_name="s")
    return pl.kernel(
        _route_body,
        out_type=jax.ShapeDtypeStruct((_NROWS, _H * _OPAD), jnp.float32),
        mesh=mesh,
        compiler_params=pltpu.CompilerParams(needs_layout_passes=False),
        scratch_types=[
            pltpu.VMEM((_H * _N,), jnp.float32),
            pltpu.VMEM((_N,), jnp.int32),
            pltpu.VMEM((1024,), jnp.int32),
            pltpu.VMEM((1024,), jnp.int32),
            pltpu.VMEM((_N,), jnp.int32),
            pltpu.VMEM((_N,), jnp.int32),
            pltpu.VMEM((32,), jnp.int32),
            pltpu.VMEM((128,), jnp.int32),
            pltpu.VMEM((128,), jnp.int32),
            pltpu.VMEM((128,), jnp.int32),
            pltpu.VMEM((_H * _OPAD,), jnp.float32),
            pltpu.VMEM((_H * 112,), jnp.float32),
            pltpu.SemaphoreType.DMA,
        ],
    )


def kernel(x, cw2, cb2, cw3, cb3, cw4, cb4, cw5, cb5, cw6, cb6, cw7, cb7,
           Wq, bq, Wk, bk, Wv, bv, Wa, ba, Wm, bm):
    NR = _B * _N
    xf = x.reshape(NR, _FEAT * _AA)
    xa = jnp.concatenate(
        [xf, jnp.zeros((NR, _XPAD - (_XCOLS - 1)), jnp.float32)], axis=1)
    Wc, biasmask = _build_conv_matrix((cw2, cw3, cw4, cw5, cw6, cw7),
                                      (cb2, cb3, cb4, cb5, cb6, cb7))
    Wqkv = jnp.concatenate([Wq.T, Wk.T, Wv.T], axis=1)
    bqkv = jnp.concatenate([bq, bk, bv])[None, :]

    BM = 512
    qkv = pl.pallas_call(
        _featqkv_body,
        grid=(NR // BM,),
        in_specs=[
            pl.BlockSpec((BM, _XPAD), lambda i: (i, 0)),
            pl.BlockSpec((_XPAD, _CCOLS), lambda i: (0, 0)),
            pl.BlockSpec((1, _CCOLS), lambda i: (0, 0)),
            pl.BlockSpec((_AFN, 3 * _HID), lambda i: (0, 0)),
            pl.BlockSpec((1, 3 * _HID), lambda i: (0, 0)),
        ],
        out_specs=pl.BlockSpec((BM, 3 * _HID), lambda i: (i, 0)),
        out_shape=jax.ShapeDtypeStruct((NR, 3 * _HID), jnp.float32),
    )(xa, Wc, biasmask, Wqkv, bqkv)

    q = qkv[:, :_HID].reshape(_B, _N, _HID)
    k = qkv[:, _HID:2 * _HID].reshape(_B, _N, _HID)
    v = qkv[:, 2 * _HID:].reshape(_B, _N, _HID)

    BS = 256
    s4 = pl.pallas_call(
        _scores_body,
        grid=(_B, _N // BS),
        in_specs=[
            pl.BlockSpec((1, BS, _HID), lambda b, i: (b, i, 0)),
            pl.BlockSpec((1, _N, _HID), lambda b, i: (b, 0, 0)),
        ],
        out_specs=pl.BlockSpec((1, BS, _H * _N), lambda b, i: (b, i, 0)),
        out_shape=jax.ShapeDtypeStruct((_B, _N, _H * _N), jnp.float32),
    )(q, k)

    probs_pad = _make_route_kernel()(s4.reshape(_NROWS, _H * _N))
    probs = probs_pad.reshape(_B, _N, _H, _OPAD)[:, :, :, :_TOPK]
    probs_e = probs.transpose(0, 2, 1, 3)[:, :, :, None, :]

    pooled_pad = pl.pallas_call(
        _pooled_body,
        in_specs=[
            pl.BlockSpec((_B, _N, _HID), lambda: (0, 0, 0)),
            pl.BlockSpec((_HID, _AFN), lambda: (0, 0)),
            pl.BlockSpec((1, _AFN), lambda: (0, 0)),
            pl.BlockSpec((_AFN, 2), lambda: (0, 0)),
            pl.BlockSpec((1, 2), lambda: (0, 0)),
        ],
        out_specs=pl.BlockSpec((8, 128), lambda: (0, 0)),
        out_shape=jax.ShapeDtypeStruct((8, 128), jnp.float32),
    )(v, Wa.T, ba[None, :], Wm.T, bm[None, :])
    pooled = pooled_pad[:_B, :2]

    return pooled, probs_e
